# Initial kernel scaffold; baseline (speedup 1.0000x reference)
import jax
import jax.numpy as jnp
from jax.experimental import pallas as pl


def kernel(hidden_states, W, e_score_correction_bias):
    raise NotImplementedError("write your pallas kernel here")


# speedup vs baseline: 6.7875x; 6.7875x over previous
import functools

import jax
import jax.numpy as jnp
from jax.experimental import pallas as pl

_N_EXPERTS = 64
_N_GROUP = 8
_EPG = 8
_TOPK_GROUP = 4
_TOP_K = 8
_SCALE = 2.5
_NEG = -1e30


def _router_block(x_ref, w_ref, bc_ref, logits_ref, idx_ref, wt_ref):
    x = x_ref[...]
    Wm = w_ref[...]
    logits = jax.lax.dot_general(
        x, Wm, (((1,), (1,)), ((), ())), preferred_element_type=jnp.float32)
    logits_ref[...] = logits
    logitsT = jax.lax.dot_general(
        Wm, x, (((1,), (1,)), ((), ())), preferred_element_type=jnp.float32)
    sT = 1.0 / (1.0 + jnp.exp(-logitsT))
    sfc = sT + bc_ref[...]

    R = sT.shape[1]
    iota8 = jax.lax.broadcasted_iota(jnp.int32, (_N_GROUP, R), 0)
    iota64 = jax.lax.broadcasted_iota(jnp.int32, (_N_EXPERTS, R), 0)

    gsums = []
    for g in range(_N_GROUP):
        v = sfc[g * _EPG:(g + 1) * _EPG, :]
        m1 = jnp.max(v, axis=0, keepdims=True)
        am = jnp.min(jnp.where(v == m1, iota8, _EPG), axis=0, keepdims=True)
        v2 = jnp.where(iota8 == am, _NEG, v)
        m2 = jnp.max(v2, axis=0, keepdims=True)
        gsums.append(m1 + m2)
    G = jnp.concatenate(gsums, axis=0)

    bids = []
    for _ in range(_TOPK_GROUP):
        m = jnp.max(G, axis=0, keepdims=True)
        bid = jnp.min(jnp.where(G == m, iota8, _N_GROUP), axis=0, keepdims=True)
        bids.append(bid)
        G = jnp.where(iota8 == bid, _NEG, G)
    keep = (iota8 == bids[0]) | (iota8 == bids[1]) | (iota8 == bids[2]) | (iota8 == bids[3])

    ms_rows = []
    for g in range(_N_GROUP):
        ms_rows.append(jnp.where(keep[g:g + 1, :], sfc[g * _EPG:(g + 1) * _EPG, :], -1.0))
    ms = jnp.concatenate(ms_rows, axis=0)

    idxs, ws = [], []
    for _ in range(_TOP_K):
        m = jnp.max(ms, axis=0, keepdims=True)
        idx = jnp.min(jnp.where(ms == m, iota64, _N_EXPERTS), axis=0, keepdims=True)
        idxs.append(idx)
        w = jnp.max(jnp.where(iota64 == idx, sT, _NEG), axis=0, keepdims=True)
        ws.append(w)
        ms = jnp.where(iota64 == idx, -2.0, ms)
    IDX = jnp.concatenate(idxs, axis=0)
    WS = jnp.concatenate(ws, axis=0)
    den = jnp.sum(WS, axis=0, keepdims=True) + 1e-20
    idx_ref[...] = IDX
    wt_ref[...] = WS * (_SCALE / den)


@functools.partial(jax.jit, static_argnames=())
def _run(x, W, bias_col):
    N, D = x.shape
    R = 512
    grid = (N // R,)
    logits, idxT, wtT = pl.pallas_call(
        _router_block,
        grid=grid,
        in_specs=[
            pl.BlockSpec((R, D), lambda i: (i, 0)),
            pl.BlockSpec((_N_EXPERTS, D), lambda i: (0, 0)),
            pl.BlockSpec((_N_EXPERTS, 1), lambda i: (0, 0)),
        ],
        out_specs=[
            pl.BlockSpec((R, _N_EXPERTS), lambda i: (i, 0)),
            pl.BlockSpec((_TOP_K, R), lambda i: (0, i)),
            pl.BlockSpec((_TOP_K, R), lambda i: (0, i)),
        ],
        out_shape=[
            jax.ShapeDtypeStruct((N, _N_EXPERTS), jnp.float32),
            jax.ShapeDtypeStruct((_TOP_K, N), jnp.int32),
            jax.ShapeDtypeStruct((_TOP_K, N), jnp.float32),
        ],
    )(x, W, bias_col)
    return logits, idxT, wtT


def kernel(hidden_states, W, e_score_correction_bias):
    B, S, D = hidden_states.shape
    N = B * S
    x = hidden_states.reshape(N, D).astype(jnp.float32)
    bias_col = e_score_correction_bias.astype(jnp.float32).reshape(_N_EXPERTS, 1)
    logits, idxT, wtT = _run(x, W.astype(jnp.float32), bias_col)
    dt = hidden_states.dtype
    return idxT.T, wtT.T.astype(dt), logits.astype(dt)
